# Initial kernel scaffold; baseline (speedup 1.0000x reference)
import jax
import jax.numpy as jnp
from jax.experimental import pallas as pl


def kernel(x):
    raise NotImplementedError("write your pallas kernel here")


# speedup vs baseline: 20.8860x; 20.8860x over previous
import jax
import jax.numpy as jnp
from jax.experimental import pallas as pl
from jax.experimental.pallas import tpu as pltpu

_ROWS = 128
_COLS = 32768
_KK = 64 * _ROWS
_CH = 8
_NCH = _ROWS // _CH

_i32 = jnp.int32
_f32 = jnp.float32


def _body(x_ref, o_ref, rc_ref):
    def init_chunk(c, carry):
        xb = x_ref[pl.ds(c * _CH, _CH), :]
        keys = jnp.maximum(jax.lax.bitcast_convert_type(xb, _i32), 0)
        o_ref[pl.ds(c * _CH, _CH), :] = jax.lax.bitcast_convert_type(keys, _f32)
        return carry

    jax.lax.fori_loop(0, _NCH, init_chunk, 0)

    def keys_chunk(c):
        return jax.lax.bitcast_convert_type(o_ref[pl.ds(c * _CH, _CH), :], _i32)

    def count_ge(t):
        def cbody(c, acc):
            return acc + jnp.sum((keys_chunk(c) >= t).astype(_i32))

        return jax.lax.fori_loop(0, _NCH, cbody, jnp.int32(0))

    kk = jnp.int32(_KK)

    def key_round(i, cur):
        cand = cur + (jnp.int32(1) << (jnp.int32(30) - i))
        return jnp.where(count_ge(cand) >= kk, cand, cur)

    kstar = jax.lax.fori_loop(0, 31, key_round, jnp.int32(0))

    count_gt = count_ge(kstar + 1)
    r = kk - count_gt

    def eqrow_chunk(c, carry):
        eqc = (keys_chunk(c) == kstar).astype(_i32)
        rc_ref[pl.ds(c * _CH, _CH), :] = jnp.sum(eqc, axis=1, keepdims=True)
        return carry

    jax.lax.fori_loop(0, _NCH, eqrow_chunk, 0)

    row_iota = jax.lax.broadcasted_iota(_i32, (_ROWS, 1), 0)

    def row_prefix(a):
        return jnp.sum(jnp.where(row_iota < a, rc_ref[...], 0))

    def row_round(i, lo):
        cand = lo + (jnp.int32(64) >> i)
        return jnp.where(row_prefix(cand) < r, cand, lo)

    brow = jax.lax.fori_loop(0, 7, row_round, jnp.int32(0))
    rem = r - row_prefix(brow)

    eq_row = (
        jax.lax.bitcast_convert_type(o_ref[pl.ds(brow, 1), :], _i32) == kstar
    ).astype(_i32)
    col_iota = jax.lax.broadcasted_iota(_i32, (1, _COLS), 1)

    def col_prefix(c):
        return jnp.sum(jnp.where(col_iota < c, eq_row, 0))

    def col_round(i, lo):
        cand = lo + (jnp.int32(16384) >> i)
        return jnp.where(col_prefix(cand) < rem, cand, lo)

    locol = jax.lax.fori_loop(0, 15, col_round, jnp.int32(0))

    def out_chunk(c, carry):
        keys = keys_chunk(c)
        rid = c * _CH + jax.lax.broadcasted_iota(_i32, (_CH, _COLS), 0)
        cid = jax.lax.broadcasted_iota(_i32, (_CH, _COLS), 1)
        keep_eq = (keys == kstar) & (
            (rid < brow) | ((rid == brow) & (cid <= locol))
        )
        keep = (keys > kstar) | keep_eq
        o_ref[pl.ds(c * _CH, _CH), :] = jnp.where(
            keep, jax.lax.bitcast_convert_type(keys, _f32), 0.0
        )
        return carry

    jax.lax.fori_loop(0, _NCH, out_chunk, 0)


def kernel(x):
    return pl.pallas_call(
        _body,
        out_shape=jax.ShapeDtypeStruct((_ROWS, _COLS), jnp.float32),
        in_specs=[pl.BlockSpec((_ROWS, _COLS), lambda: (0, 0))],
        out_specs=pl.BlockSpec((_ROWS, _COLS), lambda: (0, 0)),
        scratch_shapes=[pltpu.VMEM((_ROWS, 1), jnp.int32)],
    )(x)


# speedup vs baseline: 28.6075x; 1.3697x over previous
import jax
import jax.numpy as jnp
from jax.experimental import pallas as pl
from jax.experimental.pallas import tpu as pltpu

_ROWS = 128
_COLS = 32768
_KK = 64 * _ROWS
_CH = 8
_NCH = _ROWS // _CH

_i32 = jnp.int32
_f32 = jnp.float32


def _body(x_ref, o_ref):
    for c in range(_NCH):
        xb = x_ref[c * _CH:(c + 1) * _CH, :]
        keys = jnp.maximum(jax.lax.bitcast_convert_type(xb, _i32), 0)
        o_ref[c * _CH:(c + 1) * _CH, :] = jax.lax.bitcast_convert_type(
            keys, _f32
        )

    def all_keys():
        return jax.lax.bitcast_convert_type(o_ref[...], _i32)

    def count_ge(t):
        return jnp.sum((all_keys() >= t).astype(_i32))

    kk = jnp.int32(_KK)

    def key_round(i, cur):
        cand = cur + (jnp.int32(1) << (jnp.int32(30) - i))
        return jnp.where(count_ge(cand) >= kk, cand, cur)

    kstar = jax.lax.fori_loop(0, 31, key_round, jnp.int32(0))

    count_gt = count_ge(kstar + 1)
    r = kk - count_gt

    rc = jnp.sum((all_keys() == kstar).astype(_i32), axis=1, keepdims=True)

    row_iota = jax.lax.broadcasted_iota(_i32, (_ROWS, 1), 0)

    def row_prefix(a):
        return jnp.sum(jnp.where(row_iota < a, rc, 0))

    def row_round(i, lo):
        cand = lo + (jnp.int32(64) >> i)
        return jnp.where(row_prefix(cand) < r, cand, lo)

    brow = jax.lax.fori_loop(0, 7, row_round, jnp.int32(0))
    rem = r - row_prefix(brow)

    eq_row = (
        jax.lax.bitcast_convert_type(o_ref[pl.ds(brow, 1), :], _i32) == kstar
    ).astype(_i32)
    col_iota = jax.lax.broadcasted_iota(_i32, (1, _COLS), 1)

    def col_prefix(c):
        return jnp.sum(jnp.where(col_iota < c, eq_row, 0))

    def col_round(i, lo):
        cand = lo + (jnp.int32(16384) >> i)
        return jnp.where(col_prefix(cand) < rem, cand, lo)

    locol = jax.lax.fori_loop(0, 15, col_round, jnp.int32(0))

    for c in range(_NCH):
        keys = jax.lax.bitcast_convert_type(
            o_ref[c * _CH:(c + 1) * _CH, :], _i32
        )
        rid = c * _CH + jax.lax.broadcasted_iota(_i32, (_CH, _COLS), 0)
        cid = jax.lax.broadcasted_iota(_i32, (_CH, _COLS), 1)
        keep_eq = (keys == kstar) & (
            (rid < brow) | ((rid == brow) & (cid <= locol))
        )
        keep = (keys > kstar) | keep_eq
        o_ref[c * _CH:(c + 1) * _CH, :] = jnp.where(
            keep, jax.lax.bitcast_convert_type(keys, _f32), 0.0
        )


def kernel(x):
    return pl.pallas_call(
        _body,
        out_shape=jax.ShapeDtypeStruct((_ROWS, _COLS), jnp.float32),
        in_specs=[pl.BlockSpec((_ROWS, _COLS), lambda: (0, 0))],
        out_specs=pl.BlockSpec((_ROWS, _COLS), lambda: (0, 0)),
    )(x)


# speedup vs baseline: 57.5754x; 2.0126x over previous
import jax
import jax.numpy as jnp
from jax.experimental import pallas as pl
from jax.experimental.pallas import tpu as pltpu

_ROWS = 128
_COLS = 32768
_KK = 64 * _ROWS
_CH = 8
_NCH = _ROWS // _CH

_i32 = jnp.int32
_f32 = jnp.float32


def _body(x_ref, o_ref):
    for c in range(_NCH):
        xb = x_ref[c * _CH:(c + 1) * _CH, :]
        keys = jnp.maximum(jax.lax.bitcast_convert_type(xb, _i32), 0)
        o_ref[c * _CH:(c + 1) * _CH, :] = jax.lax.bitcast_convert_type(
            keys, _f32
        )

    def all_keys():
        return jax.lax.bitcast_convert_type(o_ref[...], _i32)

    def count_ge(t):
        vacc = jnp.zeros((_CH, 128), _i32)
        for c in range(_NCH):
            for s in range(_COLS // 4096):
                k = jax.lax.bitcast_convert_type(
                    o_ref[c * _CH:(c + 1) * _CH, s * 4096:(s + 1) * 4096],
                    _i32,
                )
                m = (k >= t).astype(_i32)
                w = 4096
                while w > 128:
                    w //= 2
                    m = m[:, :w] + m[:, w:]
                vacc = vacc + m
        return jnp.sum(vacc)

    kk = jnp.int32(_KK)

    def key_round(i, cur):
        cand = cur + (jnp.int32(1) << (jnp.int32(30) - i))
        return jnp.where(count_ge(cand) >= kk, cand, cur)

    kstar = jax.lax.fori_loop(0, 31, key_round, jnp.int32(0))

    count_gt = count_ge(kstar + 1)
    r = kk - count_gt

    rc = jnp.sum((all_keys() == kstar).astype(_i32), axis=1, keepdims=True)

    row_iota = jax.lax.broadcasted_iota(_i32, (_ROWS, 1), 0)

    def row_prefix(a):
        return jnp.sum(jnp.where(row_iota < a, rc, 0))

    def row_round(i, lo):
        cand = lo + (jnp.int32(64) >> i)
        return jnp.where(row_prefix(cand) < r, cand, lo)

    brow = jax.lax.fori_loop(0, 7, row_round, jnp.int32(0))
    rem = r - row_prefix(brow)

    eq_row = (
        jax.lax.bitcast_convert_type(o_ref[pl.ds(brow, 1), :], _i32) == kstar
    ).astype(_i32)
    col_iota = jax.lax.broadcasted_iota(_i32, (1, _COLS), 1)

    def col_prefix(c):
        return jnp.sum(jnp.where(col_iota < c, eq_row, 0))

    def col_round(i, lo):
        cand = lo + (jnp.int32(16384) >> i)
        return jnp.where(col_prefix(cand) < rem, cand, lo)

    locol = jax.lax.fori_loop(0, 15, col_round, jnp.int32(0))

    for c in range(_NCH):
        keys = jax.lax.bitcast_convert_type(
            o_ref[c * _CH:(c + 1) * _CH, :], _i32
        )
        rid = c * _CH + jax.lax.broadcasted_iota(_i32, (_CH, _COLS), 0)
        cid = jax.lax.broadcasted_iota(_i32, (_CH, _COLS), 1)
        keep_eq = (keys == kstar) & (
            (rid < brow) | ((rid == brow) & (cid <= locol))
        )
        keep = (keys > kstar) | keep_eq
        o_ref[c * _CH:(c + 1) * _CH, :] = jnp.where(
            keep, jax.lax.bitcast_convert_type(keys, _f32), 0.0
        )


def kernel(x):
    return pl.pallas_call(
        _body,
        out_shape=jax.ShapeDtypeStruct((_ROWS, _COLS), jnp.float32),
        in_specs=[pl.BlockSpec((_ROWS, _COLS), lambda: (0, 0))],
        out_specs=pl.BlockSpec((_ROWS, _COLS), lambda: (0, 0)),
    )(x)


# speedup vs baseline: 59.8348x; 1.0392x over previous
import jax
import jax.numpy as jnp
from jax.experimental import pallas as pl
from jax.experimental.pallas import tpu as pltpu

_ROWS = 128
_COLS = 32768
_TOTAL = _ROWS * _COLS
_KK = 64 * _ROWS
_CH = 8
_NCH = _ROWS // _CH
_SUB = 4096
_NSUB = _COLS // _SUB

_i32 = jnp.int32
_f32 = jnp.float32


def _lt(k, t):
    return jax.lax.shift_right_logical(k - t, 31)


def _tree(m):
    w = m.shape[1]
    while w > 128:
        w //= 2
        m = m[:, :w] + m[:, w:]
    return m


def _body(x_ref, o_ref):
    for c in range(_NCH):
        xb = x_ref[c * _CH:(c + 1) * _CH, :]
        keys = jnp.maximum(jax.lax.bitcast_convert_type(xb, _i32), 0)
        o_ref[c * _CH:(c + 1) * _CH, :] = jax.lax.bitcast_convert_type(
            keys, _f32
        )

    def kchunk(c, s=None):
        if s is None:
            sl = slice(None)
        else:
            sl = slice(s * _SUB, (s + 1) * _SUB)
        return jax.lax.bitcast_convert_type(
            o_ref[c * _CH:(c + 1) * _CH, sl], _i32
        )

    def count_lt(t):
        vacc = jnp.zeros((_CH, 128), _i32)
        for c in range(_NCH):
            for s in range(_NSUB):
                vacc = vacc + _tree(_lt(kchunk(c, s), t))
        return jnp.sum(vacc)

    kk = jnp.int32(_KK)
    ge_kk = jnp.int32(_TOTAL - _KK)

    def key_round(i, cur):
        cand = cur + (jnp.int32(1) << (jnp.int32(30) - i))
        return jnp.where(count_lt(cand) <= ge_kk, cand, cur)

    kstar = jax.lax.fori_loop(0, 31, key_round, jnp.int32(0))

    vacc = jnp.zeros((_CH, 128), _i32)
    rows = []
    for c in range(_NCH):
        racc = jnp.zeros((_CH, 128), _i32)
        for s in range(_NSUB):
            k = kchunk(c, s)
            le = _lt(k, kstar + 1)
            vacc = vacc + _tree(le)
            racc = racc + _tree(le - _lt(k, kstar))
        rows.append(jnp.sum(racc, axis=1, keepdims=True))
    count_gt = jnp.int32(_TOTAL) - jnp.sum(vacc)
    rc = jnp.concatenate(rows, axis=0)
    r = kk - count_gt

    row_iota = jax.lax.broadcasted_iota(_i32, (_ROWS, 1), 0)

    def row_prefix(a):
        return jnp.sum(jnp.where(row_iota < a, rc, 0))

    def row_round(i, lo):
        cand = lo + (jnp.int32(64) >> i)
        return jnp.where(row_prefix(cand) < r, cand, lo)

    brow = jax.lax.fori_loop(0, 7, row_round, jnp.int32(0))
    rem = r - row_prefix(brow)

    eq_row = (
        jax.lax.bitcast_convert_type(o_ref[pl.ds(brow, 1), :], _i32) == kstar
    ).astype(_i32)
    col_iota = jax.lax.broadcasted_iota(_i32, (1, _COLS), 1)

    def col_prefix(c):
        return jnp.sum(jnp.where(col_iota < c, eq_row, 0))

    def col_round(i, lo):
        cand = lo + (jnp.int32(16384) >> i)
        return jnp.where(col_prefix(cand) < rem, cand, lo)

    locol = jax.lax.fori_loop(0, 15, col_round, jnp.int32(0))

    cut = jnp.where(
        row_iota < brow,
        jnp.int32(_COLS),
        jnp.where(row_iota == brow, locol + 1, jnp.int32(0)),
    )

    for c in range(_NCH):
        k = kchunk(c)
        cid = jax.lax.broadcasted_iota(_i32, (_CH, _COLS), 1)
        cutc = cut[c * _CH:(c + 1) * _CH, :]
        keep = (k > kstar) | ((k == kstar) & (cid < cutc))
        o_ref[c * _CH:(c + 1) * _CH, :] = jnp.where(
            keep, jax.lax.bitcast_convert_type(k, _f32), 0.0
        )


def kernel(x):
    return pl.pallas_call(
        _body,
        out_shape=jax.ShapeDtypeStruct((_ROWS, _COLS), jnp.float32),
        in_specs=[pl.BlockSpec((_ROWS, _COLS), lambda: (0, 0))],
        out_specs=pl.BlockSpec((_ROWS, _COLS), lambda: (0, 0)),
    )(x)
